# baseline (speedup 1.0000x reference)
import functools

import jax
import jax.numpy as jnp
from jax import lax
from jax.experimental import pallas as pl
from jax.experimental.pallas import tpu as pltpu
from jax.experimental.pallas import tpu_sc as plsc

_VOCAB = 1000000
_EMSIZE = 64
_PADE = 128
_BATCH = 4096
_SEQLEN = 200

_N = _BATCH * _SEQLEN

_info = plsc.get_sparse_core_info()
_NC, _NS = _info.num_cores, _info.num_subcores
_NW = _NC * _NS
_RPW = _N // _NW

_K = 128
_UPW = _RPW // _K
_BBLK = _BATCH // _K


def _make_gather():
    mesh = plsc.VectorSubcoreMesh(core_axis_name="c", subcore_axis_name="s")

    @functools.partial(
        pl.kernel,
        mesh=mesh,
        out_type=jax.ShapeDtypeStruct((_SEQLEN, _EMSIZE, _BATCH), jnp.float32),
        scratch_types=[
            pltpu.VMEM((_RPW,), jnp.int32),
            [pltpu.VMEM((_K, _PADE), jnp.float32) for _ in range(2)],
            [pltpu.VMEM((_EMSIZE, _K), jnp.float32) for _ in range(2)],
            pltpu.SemaphoreType.DMA((2,)),
            pltpu.SemaphoreType.DMA((2,)),
        ],
        compiler_params=pltpu.CompilerParams(
            use_tc_tiling_on_sc=True, needs_layout_passes=False
        ),
    )
    def gather_kernel(table_hbm, idx_hbm, out_hbm, idx_v, bufs, stgs, gsem, wsem):
        wid = lax.axis_index("s") * _NC + lax.axis_index("c")
        row0 = wid * _RPW
        gu0 = wid * _UPW
        pltpu.sync_copy(idx_hbm.at[pl.ds(row0, _RPW)], idx_v)

        rows16 = [jnp.arange(jb * 16, jb * 16 + 16, dtype=jnp.int32)
                  for jb in range(_K // 16)]

        def transpose_unit(b):
            def dbody(d, carry):
                cols = jnp.full((16,), d, jnp.int32)
                for jb in range(_K // 16):
                    v = plsc.load_gather(bufs[b], [rows16[jb], cols])
                    stgs[b][d, pl.ds(jb * 16, 16)] = v
                return carry

            lax.fori_loop(0, _EMSIZE, dbody, 0)

        def unit_dst(u):
            gu = gu0 + u
            l = gu // _BBLK
            b0 = (gu % _BBLK) * _K
            return out_hbm.at[l, :, pl.ds(b0, _K)]

        for b in range(2):
            pltpu.async_copy(
                table_hbm.at[idx_v.at[pl.ds(b * _K, _K)]], bufs[b], gsem.at[b]
            )
        for b in range(2):
            pltpu.make_async_copy(
                table_hbm.at[idx_v.at[pl.ds(0, _K)]], bufs[b], gsem.at[b]
            ).wait()
            transpose_unit(b)
            pltpu.async_copy(stgs[b], unit_dst(b), wsem.at[b])
            pltpu.async_copy(
                table_hbm.at[idx_v.at[pl.ds((b + 2) * _K, _K)]],
                bufs[b],
                gsem.at[b],
            )

        def body(i, carry):
            for b in range(2):
                u = 2 * i + b
                pltpu.make_async_copy(
                    table_hbm.at[idx_v.at[pl.ds(0, _K)]], bufs[b], gsem.at[b]
                ).wait()
                pltpu.make_async_copy(
                    stgs[b], unit_dst(0), wsem.at[b]
                ).wait()
                transpose_unit(b)
                pltpu.async_copy(stgs[b], unit_dst(u), wsem.at[b])
                un = jnp.minimum(u + 2, _UPW - 1)
                pltpu.async_copy(
                    table_hbm.at[idx_v.at[pl.ds(un * _K, _K)]],
                    bufs[b],
                    gsem.at[b],
                )
            return carry

        lax.fori_loop(1, _UPW // 2, body, 0)

        for b in range(2):
            pltpu.make_async_copy(
                table_hbm.at[idx_v.at[pl.ds(0, _K)]], bufs[b], gsem.at[b]
            ).wait()
            pltpu.make_async_copy(stgs[b], unit_dst(0), wsem.at[b]).wait()

    return gather_kernel


_gather = _make_gather()


_TBC = 38400


def _make_padder():
    def body(in_ref, out_ref):
        out_ref[:, :_EMSIZE] = jnp.transpose(in_ref[...])

    return pl.pallas_call(
        body,
        grid=((_VOCAB + _TBC - 1) // _TBC,),
        in_specs=[pl.BlockSpec((_EMSIZE, _TBC), lambda i: (0, i))],
        out_specs=pl.BlockSpec((_TBC, _PADE), lambda i: (i, 0)),
        out_shape=jax.ShapeDtypeStruct((_VOCAB, _PADE), jnp.float32),
        compiler_params=pltpu.CompilerParams(vmem_limit_bytes=128 * 1024 * 1024),
    )


_padder = _make_padder()


def kernel(sequence, table):
    idx = sequence.T.astype(jnp.int32).reshape(_N)
    table_pad = _padder(table.T)
    out3 = _gather(table_pad, idx)
    return out3.transpose(2, 0, 1)


# speedup vs baseline: 1.9943x; 1.9943x over previous
import functools

import jax
import jax.numpy as jnp
from jax import lax
from jax.experimental import pallas as pl
from jax.experimental.pallas import tpu as pltpu
from jax.experimental.pallas import tpu_sc as plsc

_VOCAB = 1000000
_EMSIZE = 64
_PADE = 128
_BATCH = 4096
_SEQLEN = 200

_N = _BATCH * _SEQLEN

_info = plsc.get_sparse_core_info()
_NC, _NS = _info.num_cores, _info.num_subcores
_NW = _NC * _NS
_RPW = _N // _NW

_K = 256
_NB = 2
_CPW = _RPW // _K


def _make_gather():
    mesh = plsc.VectorSubcoreMesh(core_axis_name="c", subcore_axis_name="s")

    @functools.partial(
        pl.kernel,
        mesh=mesh,
        out_type=jax.ShapeDtypeStruct((_N, _PADE), jnp.float32),
        scratch_types=[
            pltpu.VMEM((_RPW,), jnp.int32),
            [pltpu.VMEM((_K, _PADE), jnp.float32) for _ in range(_NB)],
            pltpu.SemaphoreType.DMA((_NB,)),
            pltpu.SemaphoreType.DMA((_NB,)),
        ],
        compiler_params=pltpu.CompilerParams(use_tc_tiling_on_sc=True),
    )
    def gather_kernel(table_hbm, idx_hbm, out_hbm, idx_v, bufs, gsem, wsem):
        wid = lax.axis_index("s") * _NC + lax.axis_index("c")
        row0 = wid * _RPW
        pltpu.sync_copy(idx_hbm.at[pl.ds(row0, _RPW)], idx_v)

        for b in range(_NB):
            pltpu.async_copy(
                table_hbm.at[idx_v.at[pl.ds(b * _K, _K)]],
                bufs[b],
                gsem.at[b],
            )

        def body(i, carry):
            for b in range(_NB):
                g = i * _NB + b
                pltpu.make_async_copy(
                    table_hbm.at[idx_v.at[pl.ds(g * _K, _K)]],
                    bufs[b],
                    gsem.at[b],
                ).wait()
                pltpu.async_copy(
                    bufs[b],
                    out_hbm.at[pl.ds(row0 + g * _K, _K)],
                    wsem.at[b],
                )
            for b in range(_NB):
                gnext = jnp.minimum((i + 1) * _NB + b, _CPW - 1)
                pltpu.make_async_copy(
                    bufs[b], out_hbm.at[pl.ds(0, _K)], wsem.at[b]
                ).wait()
                pltpu.async_copy(
                    table_hbm.at[idx_v.at[pl.ds(gnext * _K, _K)]],
                    bufs[b],
                    gsem.at[b],
                )
            return carry

        lax.fori_loop(0, _CPW // _NB, body, 0)

        for b in range(_NB):
            pltpu.make_async_copy(
                table_hbm.at[idx_v.at[pl.ds(0, _K)]],
                bufs[b],
                gsem.at[b],
            ).wait()

    return gather_kernel


_gather = _make_gather()


_TBC = 38400


def _make_padder():
    def body(in_ref, out_ref):
        out_ref[:, :_EMSIZE] = jnp.transpose(in_ref[...])

    return pl.pallas_call(
        body,
        grid=((_VOCAB + _TBC - 1) // _TBC,),
        in_specs=[pl.BlockSpec((_EMSIZE, _TBC), lambda i: (0, i))],
        out_specs=pl.BlockSpec((_TBC, _PADE), lambda i: (i, 0)),
        out_shape=jax.ShapeDtypeStruct((_VOCAB, _PADE), jnp.float32),
        compiler_params=pltpu.CompilerParams(vmem_limit_bytes=128 * 1024 * 1024),
    )


_padder = _make_padder()


def kernel(sequence, table):
    idx = sequence.astype(jnp.int32).reshape(_N)
    table_pad = _padder(table.T)
    out = _gather(table_pad, idx)
    return out[:, :_EMSIZE].reshape(_BATCH, _SEQLEN, _EMSIZE)


# speedup vs baseline: 2.0010x; 1.0034x over previous
import functools

import jax
import jax.numpy as jnp
from jax import lax
from jax.experimental import pallas as pl
from jax.experimental.pallas import tpu as pltpu
from jax.experimental.pallas import tpu_sc as plsc

_VOCAB = 1000000
_EMSIZE = 64
_PADE = 128
_BATCH = 4096
_SEQLEN = 200

_N = _BATCH * _SEQLEN

_info = plsc.get_sparse_core_info()
_NC, _NS = _info.num_cores, _info.num_subcores
_NW = _NC * _NS
_RPW = _N // _NW

_K = 128
_NB = 4
_CPW = _RPW // _K


def _make_gather():
    mesh = plsc.VectorSubcoreMesh(core_axis_name="c", subcore_axis_name="s")

    @functools.partial(
        pl.kernel,
        mesh=mesh,
        out_type=jax.ShapeDtypeStruct((_N, _PADE), jnp.float32),
        scratch_types=[
            pltpu.VMEM((_RPW,), jnp.int32),
            [pltpu.VMEM((_K, _PADE), jnp.float32) for _ in range(_NB)],
            pltpu.SemaphoreType.DMA((_NB,)),
            pltpu.SemaphoreType.DMA((_NB,)),
        ],
        compiler_params=pltpu.CompilerParams(use_tc_tiling_on_sc=True),
    )
    def gather_kernel(table_hbm, idx_hbm, out_hbm, idx_v, bufs, gsem, wsem):
        wid = lax.axis_index("s") * _NC + lax.axis_index("c")
        row0 = wid * _RPW
        pltpu.sync_copy(idx_hbm.at[pl.ds(row0, _RPW)], idx_v)

        for b in range(_NB):
            pltpu.async_copy(
                table_hbm.at[idx_v.at[pl.ds(b * _K, _K)]],
                bufs[b],
                gsem.at[b],
            )

        def body(i, carry):
            for b in range(_NB):
                g = i * _NB + b
                pltpu.make_async_copy(
                    table_hbm.at[idx_v.at[pl.ds(g * _K, _K)]],
                    bufs[b],
                    gsem.at[b],
                ).wait()
                pltpu.async_copy(
                    bufs[b],
                    out_hbm.at[pl.ds(row0 + g * _K, _K)],
                    wsem.at[b],
                )
            for b in range(_NB):
                gnext = jnp.minimum((i + 1) * _NB + b, _CPW - 1)
                pltpu.make_async_copy(
                    bufs[b], out_hbm.at[pl.ds(0, _K)], wsem.at[b]
                ).wait()
                pltpu.async_copy(
                    table_hbm.at[idx_v.at[pl.ds(gnext * _K, _K)]],
                    bufs[b],
                    gsem.at[b],
                )
            return carry

        lax.fori_loop(0, _CPW // _NB, body, 0)

        for b in range(_NB):
            pltpu.make_async_copy(
                table_hbm.at[idx_v.at[pl.ds(0, _K)]],
                bufs[b],
                gsem.at[b],
            ).wait()

    return gather_kernel


_gather = _make_gather()


_TBC = 38400


def _make_padder():
    def body(in_ref, out_ref):
        out_ref[:, :_EMSIZE] = jnp.transpose(in_ref[...])

    return pl.pallas_call(
        body,
        grid=((_VOCAB + _TBC - 1) // _TBC,),
        in_specs=[pl.BlockSpec((_EMSIZE, _TBC), lambda i: (0, i))],
        out_specs=pl.BlockSpec((_TBC, _PADE), lambda i: (i, 0)),
        out_shape=jax.ShapeDtypeStruct((_VOCAB, _PADE), jnp.float32),
        compiler_params=pltpu.CompilerParams(vmem_limit_bytes=128 * 1024 * 1024),
    )


_padder = _make_padder()


def kernel(sequence, table):
    idx = sequence.astype(jnp.int32).reshape(_N)
    table_pad = _padder(table.T)
    out = _gather(table_pad, idx)
    return out[:, :_EMSIZE].reshape(_BATCH, _SEQLEN, _EMSIZE)
